# Initial kernel scaffold; baseline (speedup 1.0000x reference)
import functools

import jax
import jax.numpy as jnp
from jax import lax
from jax.experimental import pallas as pl
from jax.experimental.pallas import tpu as pltpu
from jax.experimental.pallas import tpu_sc as plsc

N = 10000
E = 160000
HID = 32
HEADS = 5
G = 64
NUM_LAYERS = 5
D_FEAT = 128

NC = 2
NS = 16
NWK = NC * NS
W = 128
NWIN = 40
EP = NWK * NWIN * W
PAD = EP - E
NACC = 10240
NSLICE = NACC // NS
BR = 2000

_HI = jax.lax.Precision.HIGHEST

SRC_OF = [0, 1, 1, 2, 1]
DST_OF = [1, 1, 2, 1, 0]
CONV_KEYS = ["input->function", "function->function", "function->output",
             "output->function", "function->input"]


def _prep_edges(ei):
    src = ei[0].astype(jnp.int32)
    dst = ei[1].astype(jnp.int32)
    pad_src = jnp.zeros((PAD,), jnp.int32)
    pad_dst = N + (jnp.arange(PAD, dtype=jnp.int32) % (NACC - N))
    srcp = jnp.concatenate([src, pad_src]).reshape(NWK, NWIN, W)
    dstp = jnp.concatenate([dst, pad_dst]).reshape(NWK, NWIN, W)
    return srcp, dstp


def _sc_counts(dsts, ones_w, zeros16):
    mesh = plsc.VectorSubcoreMesh(core_axis_name="c", subcore_axis_name="s")
    n = len(dsts)
    out_type = [jax.ShapeDtypeStruct((NC, NACC, 16), jnp.float32)
                for _ in range(n)]
    scratch_types = [
        pltpu.VMEM((NWIN, W), jnp.int32),
        pltpu.VMEM((W, 16), jnp.float32),
        pltpu.VMEM((NSLICE, 16), jnp.float32),
        pltpu.VMEM_SHARED((NACC, 16), jnp.float32),
    ]

    @functools.partial(pl.kernel, mesh=mesh, out_type=out_type,
                       scratch_types=scratch_types)
    def k(ones_ref, zeros_ref, *refs):
        d_refs = refs[0:n]
        out_refs = refs[n:2 * n]
        dbuf, ones_v, zbuf, acc_sh = refs[2 * n:]
        cid = lax.axis_index("c")
        sid = lax.axis_index("s")
        wid = cid * NS + sid
        pltpu.sync_copy(ones_ref, ones_v)
        pltpu.sync_copy(zeros_ref, zbuf)
        for t in range(n):
            pltpu.sync_copy(zbuf, acc_sh.at[pl.ds(sid * NSLICE, NSLICE)])
            pltpu.sync_copy(d_refs[t].at[wid], dbuf)
            plsc.subcore_barrier()

            @pl.loop(0, NWIN)
            def _(w):
                pltpu.sync_copy(ones_v, acc_sh.at[dbuf.at[w]], add=True)

            plsc.subcore_barrier()
            pltpu.sync_copy(acc_sh.at[pl.ds(sid * NSLICE, NSLICE)],
                            out_refs[t].at[cid].at[pl.ds(sid * NSLICE, NSLICE)])
            if t + 1 < n:
                plsc.subcore_barrier()

    return k(ones_w, zeros16, *dsts)


def _sc_segsum5(hs, srcs, dsts, zeros32):
    mesh = plsc.VectorSubcoreMesh(core_axis_name="c", subcore_axis_name="s")
    n = len(hs)
    out_type = [jax.ShapeDtypeStruct((NC, NACC, HID), jnp.float32)
                for _ in range(n)]
    scratch_types = [
        pltpu.VMEM((NWIN, W), jnp.int32),
        pltpu.VMEM((NWIN, W), jnp.int32),
        pltpu.VMEM((W, HID), jnp.float32),
        pltpu.VMEM((NSLICE, HID), jnp.float32),
        pltpu.VMEM_SHARED((NACC, HID), jnp.float32),
    ]

    @functools.partial(pl.kernel, mesh=mesh, out_type=out_type,
                       scratch_types=scratch_types)
    def k(zeros_ref, *refs):
        h_refs = refs[0:n]
        s_refs = refs[n:2 * n]
        d_refs = refs[2 * n:3 * n]
        out_refs = refs[3 * n:4 * n]
        sbuf, dbuf, rows, zbuf, acc_sh = refs[4 * n:]
        cid = lax.axis_index("c")
        sid = lax.axis_index("s")
        wid = cid * NS + sid
        pltpu.sync_copy(zeros_ref, zbuf)
        for c in range(n):
            pltpu.sync_copy(zbuf, acc_sh.at[pl.ds(sid * NSLICE, NSLICE)])
            pltpu.sync_copy(s_refs[c].at[wid], sbuf)
            pltpu.sync_copy(d_refs[c].at[wid], dbuf)
            plsc.subcore_barrier()

            @pl.loop(0, NWIN)
            def _(w):
                pltpu.sync_copy(h_refs[c].at[sbuf.at[w]], rows)
                pltpu.sync_copy(rows, acc_sh.at[dbuf.at[w]], add=True)

            plsc.subcore_barrier()
            pltpu.sync_copy(acc_sh.at[pl.ds(sid * NSLICE, NSLICE)],
                            out_refs[c].at[cid].at[pl.ds(sid * NSLICE, NSLICE)])
            if c + 1 < n:
                plsc.subcore_barrier()

    return k(zeros32, *hs, *srcs, *dsts)


def _tc_project(xs, Wls, Wrs, brs, din):
    def body(*refs):
        x = refs[0:3]
        wl = refs[3:8]
        wr = refs[8:13]
        br = refs[13:18]
        h_out = refs[18:23]
        r_out = refs[23:28]
        for c in range(5):
            h_out[c][...] = jnp.dot(x[SRC_OF[c]][...], wl[c][...],
                                    precision=_HI)
            r_out[c][...] = jnp.dot(x[DST_OF[c]][...], wr[c][...],
                                    precision=_HI) + br[c][...]

    xspec = pl.BlockSpec((BR, din), lambda g: (g, 0))
    wspec = pl.BlockSpec((din, HID), lambda g: (0, 0))
    bspec = pl.BlockSpec((1, HID), lambda g: (0, 0))
    ospec = pl.BlockSpec((BR, HID), lambda g: (g, 0))
    out_shape = [jax.ShapeDtypeStruct((N, HID), jnp.float32)] * 10
    outs = pl.pallas_call(
        body,
        grid=(N // BR,),
        in_specs=[xspec] * 3 + [wspec] * 10 + [bspec] * 5,
        out_specs=[ospec] * 10,
        out_shape=out_shape,
    )(*xs, *Wls, *Wrs, *brs)
    return outs[0:5], outs[5:10]


def _tc_combine(aggs, cnts, rs, bls, ln_g, ln_b):
    def body(*refs):
        agg = refs[0:5]
        cnt = refs[5:10]
        r = refs[10:15]
        bl = refs[15:20]
        g_ref, b_ref = refs[20], refs[21]
        outs = refs[22:25]
        acc = [None, None, None]
        for c in range(5):
            a3 = agg[c][...]
            a = a3[0] + a3[1]
            c3 = cnt[c][...]
            cn = (c3[0] + c3[1])[:, 0:1]
            t = a / jnp.maximum(cn, 1.0) + bl[c][...] + r[c][...]
            nrm = jnp.sqrt(jnp.sum(t * t, axis=-1, keepdims=True))
            t = t / jnp.maximum(nrm, 1e-12)
            d = DST_OF[c]
            slot = {1: 0, 2: 1, 0: 2}[d]
            acc[slot] = t if acc[slot] is None else acc[slot] + t
        gv = g_ref[...]
        bv = b_ref[...]
        for i in range(3):
            v = acc[i]
            v = jnp.where(v > 0, v, jnp.expm1(v))
            m = jnp.mean(v, axis=-1, keepdims=True)
            dlt = v - m
            var = jnp.mean(dlt * dlt, axis=-1, keepdims=True)
            outs[i][...] = dlt / jnp.sqrt(var + 1e-5) * gv + bv

    aspec = pl.BlockSpec((NC, BR, HID), lambda g: (0, g, 0))
    cspec = pl.BlockSpec((NC, BR, 16), lambda g: (0, g, 0))
    rspec = pl.BlockSpec((BR, HID), lambda g: (g, 0))
    bspec = pl.BlockSpec((1, HID), lambda g: (0, 0))
    ospec = pl.BlockSpec((BR, HID), lambda g: (g, 0))
    out_shape = [jax.ShapeDtypeStruct((N, HID), jnp.float32)] * 3
    return pl.pallas_call(
        body,
        grid=(N // BR,),
        in_specs=[aspec] * 5 + [cspec] * 5 + [rspec] * 5 + [bspec] * 7,
        out_specs=[ospec] * 3,
        out_shape=out_shape,
    )(*aggs, *cnts, *rs, *bls, ln_g, ln_b)


def _tc_final(x_fn, batch2d, Wa, ba, mu_w, pi_w):
    def body(x_ref, b_ref, wa_ref, ba_ref,
             mw1, mb1, mw2, mb2, mw3, mb3,
             pw1, pb1, pw2, pb2, pw3, pb3,
             lcb_ref, ucb_ref, mu_ref):
        x = x_ref[...]
        s = jnp.dot(x, wa_ref[...], precision=_HI) + ba_ref[...]
        m = jnp.max(s, axis=0, keepdims=True)
        e = jnp.exp(s - m)
        att = e / jnp.sum(e, axis=0, keepdims=True)
        sel = (lax.broadcasted_iota(jnp.int32, (G, N), 0)
               == b_ref[...]).astype(jnp.float32)
        bm = jnp.concatenate([x * att[:, i:i + 1] for i in range(HEADS)],
                             axis=1)
        feat = jnp.dot(sel, bm, precision=_HI)

        def mlp(z, w1, b1, w2, b2, w3, b3):
            z = jnp.dot(z, w1[...], precision=_HI) + b1[...]
            z = jnp.where(z > 0, z, jnp.expm1(z))
            z = jnp.dot(z, w2[...], precision=_HI) + b2[...]
            z = jnp.where(z > 0, z, jnp.expm1(z))
            return jnp.dot(z, w3[...], precision=_HI) + b3[...]

        mu = mlp(feat, mw1, mb1, mw2, mb2, mw3, mb3)
        pi = mlp(feat, pw1, pb1, pw2, pb2, pw3, pb3)
        mu_ref[...] = mu.reshape(1, G)
        lcb_ref[...] = mu.reshape(1, G) - jnp.exp(pi[:, 0].reshape(1, G))
        ucb_ref[...] = mu.reshape(1, G) + jnp.exp(pi[:, 1].reshape(1, G))

    def full(shape):
        nd = len(shape)
        return pl.BlockSpec(shape, lambda: (0,) * nd)

    (mw1, mb1), (mw2, mb2), (mw3, mb3) = mu_w
    (pw1, pb1), (pw2, pb2), (pw3, pb3) = pi_w
    args = [x_fn, batch2d, Wa, ba.reshape(1, HEADS),
            mw1, mb1.reshape(1, -1), mw2, mb2.reshape(1, -1),
            mw3, mb3.reshape(1, -1),
            pw1, pb1.reshape(1, -1), pw2, pb2.reshape(1, -1),
            pw3, pb3.reshape(1, -1)]
    in_specs = [full(a.shape) for a in args]
    out_shape = [jax.ShapeDtypeStruct((1, G), jnp.float32)] * 3
    out_specs = [full((1, G))] * 3
    lcb, ucb, mu = pl.pallas_call(
        body,
        in_specs=in_specs,
        out_specs=out_specs,
        out_shape=out_shape,
    )(*args)
    return lcb.reshape(G), ucb.reshape(G), mu.reshape(G, 1)


def kernel(x_input, x_function, x_output, edge_index_input_function,
           edge_index_function_function, edge_index_function_output,
           edge_index_output_function, edge_index_function_input,
           batch, params):
    eis = [edge_index_input_function, edge_index_function_function,
           edge_index_function_output, edge_index_output_function,
           edge_index_function_input]
    preps = [_prep_edges(e) for e in eis]
    srcs = [p[0] for p in preps]
    dsts = [p[1] for p in preps]

    ones_w = jnp.ones((W, 16), jnp.float32)
    zeros16 = jnp.zeros((NSLICE, 16), jnp.float32)
    zeros32 = jnp.zeros((NSLICE, HID), jnp.float32)
    cnts = _sc_counts(dsts, ones_w, zeros16)

    ln_g = params["ln_g"].reshape(1, HID)
    ln_b = params["ln_b"].reshape(1, HID)
    xs = [x_input.astype(jnp.float32), x_function.astype(jnp.float32),
          x_output.astype(jnp.float32)]
    for l in range(NUM_LAYERS):
        lp = params["convs"][l]
        Wls = [lp[k]["Wl"] for k in CONV_KEYS]
        bls = [lp[k]["bl"].reshape(1, HID) for k in CONV_KEYS]
        Wrs = [lp[k]["Wr"] for k in CONV_KEYS]
        brs = [lp[k]["br"].reshape(1, HID) for k in CONV_KEYS]
        din = D_FEAT if l == 0 else HID
        hs, rs = _tc_project(xs, Wls, Wrs, brs, din)
        aggs = _sc_segsum5(list(hs), srcs, dsts, zeros32)
        xfn, xout, xin = _tc_combine(list(aggs), cnts, list(rs), bls,
                                     ln_g, ln_b)
        xs = [xin, xfn, xout]

    batch2d = batch.astype(jnp.int32).reshape(1, N)
    Wa, ba = params["att"]
    return _tc_final(xs[1], batch2d, Wa, ba, params["mu"], params["pi"])


# speedup vs baseline: 4.0414x; 4.0414x over previous
import functools

import jax
import jax.numpy as jnp
from jax import lax
from jax.experimental import pallas as pl
from jax.experimental.pallas import tpu as pltpu
from jax.experimental.pallas import tpu_sc as plsc

N = 10000
E = 160000
HID = 32
HEADS = 5
G = 64
NUM_LAYERS = 5
D_FEAT = 128

NC = 2
NS = 16
NWK = NC * NS
W = 128
NWIN = 40
EP = NWK * NWIN * W
PAD = EP - E
NACC = 10240
NSLICE = NACC // NS
BR = 1000

_HI = jax.lax.Precision.HIGHEST

SRC_OF = [0, 1, 1, 2, 1]
DST_OF = [1, 1, 2, 1, 0]
CONV_KEYS = ["input->function", "function->function", "function->output",
             "output->function", "function->input"]

_SC_PARAMS = pltpu.CompilerParams(use_tc_tiling_on_sc=False)


def _prep_edges(ei):
    src = ei[0].astype(jnp.int32)
    dst = ei[1].astype(jnp.int32)
    pad_src = jnp.zeros((PAD,), jnp.int32)
    pad_dst = N + (jnp.arange(PAD, dtype=jnp.int32) % (NACC - N))
    srcp = jnp.concatenate([src, pad_src]).reshape(NWK, NWIN, W)
    dstp = jnp.concatenate([dst, pad_dst]).reshape(NWK, NWIN, W)
    return srcp, dstp


def _sc_counts(dsts, ones_w, zeros16):
    mesh = plsc.VectorSubcoreMesh(core_axis_name="c", subcore_axis_name="s")
    n = len(dsts)
    out_type = [jax.ShapeDtypeStruct((NC, NACC, 16), jnp.float32)
                for _ in range(n)]
    scratch_types = [
        pltpu.VMEM((NWIN, W), jnp.int32),
        pltpu.VMEM((W, 16), jnp.float32),
        pltpu.VMEM((NSLICE, 16), jnp.float32),
        pltpu.VMEM_SHARED((NACC, 16), jnp.float32),
    ]

    @functools.partial(pl.kernel, mesh=mesh, out_type=out_type,
                       scratch_types=scratch_types,
                       compiler_params=_SC_PARAMS)
    def k(ones_ref, zeros_ref, *refs):
        d_refs = refs[0:n]
        out_refs = refs[n:2 * n]
        dbuf, ones_v, zbuf, acc_sh = refs[2 * n:]
        cid = lax.axis_index("c")
        sid = lax.axis_index("s")
        wid = cid * NS + sid
        pltpu.sync_copy(ones_ref, ones_v)
        pltpu.sync_copy(zeros_ref, zbuf)
        for t in range(n):
            pltpu.sync_copy(zbuf, acc_sh.at[pl.ds(sid * NSLICE, NSLICE)])
            pltpu.sync_copy(d_refs[t].at[wid], dbuf)
            plsc.subcore_barrier()

            @pl.loop(0, NWIN)
            def _(w):
                pltpu.sync_copy(ones_v, acc_sh.at[dbuf.at[w]], add=True)

            plsc.subcore_barrier()
            pltpu.sync_copy(acc_sh.at[pl.ds(sid * NSLICE, NSLICE)],
                            out_refs[t].at[cid].at[pl.ds(sid * NSLICE, NSLICE)])
            if t + 1 < n:
                plsc.subcore_barrier()

    return k(ones_w, zeros16, *dsts)


def _sc_segsum5(hs, srcs, dsts, zeros32):
    mesh = plsc.VectorSubcoreMesh(core_axis_name="c", subcore_axis_name="s")
    n = len(hs)
    out_type = [jax.ShapeDtypeStruct((NC, NACC, HID), jnp.float32)
                for _ in range(n)]
    scratch_types = [
        pltpu.VMEM((NWIN, W), jnp.int32),
        pltpu.VMEM((NWIN, W), jnp.int32),
        pltpu.VMEM((W, HID), jnp.float32),
        pltpu.VMEM((NSLICE, HID), jnp.float32),
        pltpu.VMEM_SHARED((NACC, HID), jnp.float32),
    ]

    @functools.partial(pl.kernel, mesh=mesh, out_type=out_type,
                       scratch_types=scratch_types,
                       compiler_params=_SC_PARAMS)
    def k(zeros_ref, *refs):
        h_refs = refs[0:n]
        s_refs = refs[n:2 * n]
        d_refs = refs[2 * n:3 * n]
        out_refs = refs[3 * n:4 * n]
        sbuf, dbuf, rows, zbuf, acc_sh = refs[4 * n:]
        cid = lax.axis_index("c")
        sid = lax.axis_index("s")
        wid = cid * NS + sid
        pltpu.sync_copy(zeros_ref, zbuf)
        for c in range(n):
            pltpu.sync_copy(zbuf, acc_sh.at[pl.ds(sid * NSLICE, NSLICE)])
            pltpu.sync_copy(s_refs[c].at[wid], sbuf)
            pltpu.sync_copy(d_refs[c].at[wid], dbuf)
            plsc.subcore_barrier()

            @pl.loop(0, NWIN)
            def _(w):
                pltpu.sync_copy(h_refs[c].at[sbuf.at[w]], rows)
                pltpu.sync_copy(rows, acc_sh.at[dbuf.at[w]], add=True)

            plsc.subcore_barrier()
            pltpu.sync_copy(acc_sh.at[pl.ds(sid * NSLICE, NSLICE)],
                            out_refs[c].at[cid].at[pl.ds(sid * NSLICE, NSLICE)])
            if c + 1 < n:
                plsc.subcore_barrier()

    return k(zeros32, *hs, *srcs, *dsts)


def _tc_project(xs, Wls, Wrs, brs, din):
    def body(*refs):
        x = refs[0:3]
        wl = refs[3:8]
        wr = refs[8:13]
        br = refs[13:18]
        h_out = refs[18:23]
        r_out = refs[23:28]
        for c in range(5):
            h_out[c][...] = jnp.dot(x[SRC_OF[c]][...], wl[c][...],
                                    precision=_HI)
            r_out[c][...] = jnp.dot(x[DST_OF[c]][...], wr[c][...],
                                    precision=_HI) + br[c][...]

    xspec = pl.BlockSpec((BR, din), lambda g: (g, 0))
    wspec = pl.BlockSpec((din, HID), lambda g: (0, 0))
    bspec = pl.BlockSpec((1, HID), lambda g: (0, 0))
    ospec = pl.BlockSpec((BR, HID), lambda g: (g, 0))
    out_shape = [jax.ShapeDtypeStruct((N, HID), jnp.float32)] * 10
    outs = pl.pallas_call(
        body,
        grid=(N // BR,),
        in_specs=[xspec] * 3 + [wspec] * 10 + [bspec] * 5,
        out_specs=[ospec] * 10,
        out_shape=out_shape,
    )(*xs, *Wls, *Wrs, *brs)
    return outs[0:5], outs[5:10]


def _tc_combine(aggs, cnts, rs, bls, ln_g, ln_b):
    def body(*refs):
        agg = refs[0:5]
        cnt = refs[5:10]
        r = refs[10:15]
        bl = refs[15:20]
        g_ref, b_ref = refs[20], refs[21]
        outs = refs[22:25]
        acc = [None, None, None]
        for c in range(5):
            a3 = agg[c][...]
            a = a3[0] + a3[1]
            c3 = cnt[c][...]
            cn = (c3[0] + c3[1])[:, 0:1]
            t = a / jnp.maximum(cn, 1.0) + bl[c][...] + r[c][...]
            nrm = jnp.sqrt(jnp.sum(t * t, axis=-1, keepdims=True))
            t = t / jnp.maximum(nrm, 1e-12)
            d = DST_OF[c]
            slot = {1: 0, 2: 1, 0: 2}[d]
            acc[slot] = t if acc[slot] is None else acc[slot] + t
        gv = g_ref[...]
        bv = b_ref[...]
        for i in range(3):
            v = acc[i]
            v = jnp.where(v > 0, v, jnp.exp(v) - 1.0)
            m = jnp.mean(v, axis=-1, keepdims=True)
            dlt = v - m
            var = jnp.mean(dlt * dlt, axis=-1, keepdims=True)
            outs[i][...] = dlt / jnp.sqrt(var + 1e-5) * gv + bv

    aspec = pl.BlockSpec((NC, BR, HID), lambda g: (0, g, 0))
    cspec = pl.BlockSpec((NC, BR, 16), lambda g: (0, g, 0))
    rspec = pl.BlockSpec((BR, HID), lambda g: (g, 0))
    bspec = pl.BlockSpec((1, HID), lambda g: (0, 0))
    ospec = pl.BlockSpec((BR, HID), lambda g: (g, 0))
    out_shape = [jax.ShapeDtypeStruct((N, HID), jnp.float32)] * 3
    return pl.pallas_call(
        body,
        grid=(N // BR,),
        in_specs=[aspec] * 5 + [cspec] * 5 + [rspec] * 5 + [bspec] * 7,
        out_specs=[ospec] * 3,
        out_shape=out_shape,
    )(*aggs, *cnts, *rs, *bls, ln_g, ln_b)


def _tc_final(x_fn, batch2d, Wa, ba, mu_w, pi_w):
    def body(x_ref, b_ref, wa_ref, ba_ref,
             mw1, mb1, mw2, mb2, mw3, mb3,
             pw1, pb1, pw2, pb2, pw3, pb3,
             lcb_ref, ucb_ref, mu_ref):
        x = x_ref[...]
        s = jnp.dot(x, wa_ref[...], precision=_HI) + ba_ref[...]
        m = jnp.max(s, axis=0, keepdims=True)
        e = jnp.exp(s - m)
        att = e / jnp.sum(e, axis=0, keepdims=True)
        sel = (lax.broadcasted_iota(jnp.int32, (G, N), 0)
               == b_ref[...]).astype(jnp.float32)
        bm = jnp.concatenate([x * att[:, i:i + 1] for i in range(HEADS)],
                             axis=1)
        feat = jnp.dot(sel, bm, precision=_HI)

        def mlp(z, w1, b1, w2, b2, w3, b3):
            z = jnp.dot(z, w1[...], precision=_HI) + b1[...]
            z = jnp.where(z > 0, z, jnp.exp(z) - 1.0)
            z = jnp.dot(z, w2[...], precision=_HI) + b2[...]
            z = jnp.where(z > 0, z, jnp.exp(z) - 1.0)
            return jnp.dot(z, w3[...], precision=_HI) + b3[...]

        mu = mlp(feat, mw1, mb1, mw2, mb2, mw3, mb3)
        pi = mlp(feat, pw1, pb1, pw2, pb2, pw3, pb3)
        mu_ref[...] = mu.reshape(1, G)
        lcb_ref[...] = mu.reshape(1, G) - jnp.exp(pi[:, 0].reshape(1, G))
        ucb_ref[...] = mu.reshape(1, G) + jnp.exp(pi[:, 1].reshape(1, G))

    def full(shape):
        nd = len(shape)
        return pl.BlockSpec(shape, lambda: (0,) * nd)

    (mw1, mb1), (mw2, mb2), (mw3, mb3) = mu_w
    (pw1, pb1), (pw2, pb2), (pw3, pb3) = pi_w
    args = [x_fn, batch2d, Wa, ba.reshape(1, HEADS),
            mw1, mb1.reshape(1, -1), mw2, mb2.reshape(1, -1),
            mw3, mb3.reshape(1, -1),
            pw1, pb1.reshape(1, -1), pw2, pb2.reshape(1, -1),
            pw3, pb3.reshape(1, -1)]
    in_specs = [full(a.shape) for a in args]
    out_shape = [jax.ShapeDtypeStruct((1, G), jnp.float32)] * 3
    out_specs = [full((1, G))] * 3
    lcb, ucb, mu = pl.pallas_call(
        body,
        in_specs=in_specs,
        out_specs=out_specs,
        out_shape=out_shape,
    )(*args)
    return lcb.reshape(G), ucb.reshape(G), mu.reshape(G, 1)


def kernel(x_input, x_function, x_output, edge_index_input_function,
           edge_index_function_function, edge_index_function_output,
           edge_index_output_function, edge_index_function_input,
           batch, params):
    eis = [edge_index_input_function, edge_index_function_function,
           edge_index_function_output, edge_index_output_function,
           edge_index_function_input]
    preps = [_prep_edges(e) for e in eis]
    srcs = [p[0] for p in preps]
    dsts = [p[1] for p in preps]

    ones_w = jnp.ones((W, 16), jnp.float32)
    zeros16 = jnp.zeros((NSLICE, 16), jnp.float32)
    zeros32 = jnp.zeros((NSLICE, HID), jnp.float32)
    cnts = _sc_counts(dsts, ones_w, zeros16)

    ln_g = params["ln_g"].reshape(1, HID)
    ln_b = params["ln_b"].reshape(1, HID)
    xs = [x_input.astype(jnp.float32), x_function.astype(jnp.float32),
          x_output.astype(jnp.float32)]
    for l in range(NUM_LAYERS):
        lp = params["convs"][l]
        Wls = [lp[k]["Wl"] for k in CONV_KEYS]
        bls = [lp[k]["bl"].reshape(1, HID) for k in CONV_KEYS]
        Wrs = [lp[k]["Wr"] for k in CONV_KEYS]
        brs = [lp[k]["br"].reshape(1, HID) for k in CONV_KEYS]
        din = D_FEAT if l == 0 else HID
        hs, rs = _tc_project(xs, Wls, Wrs, brs, din)
        aggs = _sc_segsum5(list(hs), srcs, dsts, zeros32)
        xfn, xout, xin = _tc_combine(list(aggs), cnts, list(rs), bls,
                                     ln_g, ln_b)
        xs = [xin, xfn, xout]

    batch2d = batch.astype(jnp.int32).reshape(1, N)
    Wa, ba = params["att"]
    return _tc_final(xs[1], batch2d, Wa, ba, params["mu"], params["pi"])


# speedup vs baseline: 4.3359x; 1.0729x over previous
import functools

import jax
import jax.numpy as jnp
from jax import lax
from jax.experimental import pallas as pl
from jax.experimental.pallas import tpu as pltpu
from jax.experimental.pallas import tpu_sc as plsc

N = 10000
E = 160000
HID = 32
HEADS = 5
G = 64
NUM_LAYERS = 5
D_FEAT = 128

NC = 2
NS = 16
NWK = NC * NS
W = 128
NWIN0 = 52
NWIN1 = 28
EP = NS * (NWIN0 + NWIN1) * W
PAD = EP - E
NACC = 10240
NSLICE = NACC // NS
BR = 1000

_HI = jax.lax.Precision.HIGHEST

SRC_OF = [0, 1, 1, 2, 1]
DST_OF = [1, 1, 2, 1, 0]
CONV_KEYS = ["input->function", "function->function", "function->output",
             "output->function", "function->input"]

_SC_PARAMS = pltpu.CompilerParams(use_tc_tiling_on_sc=False)


def _prep_edges(ei):
    src = ei[0].astype(jnp.int32)
    dst = ei[1].astype(jnp.int32)
    pad_src = jnp.zeros((PAD,), jnp.int32)
    pad_dst = N + (jnp.arange(PAD, dtype=jnp.int32) % (NACC - N))

    def layout(flat):
        e0 = NS * NWIN0 * W
        c0 = flat[:e0].reshape(NS, NWIN0, W)
        c1 = flat[e0:].reshape(NS, NWIN1, W)
        c1 = jnp.pad(c1, ((0, 0), (0, NWIN0 - NWIN1), (0, 0)))
        return jnp.concatenate([c0, c1], axis=0)

    srcp = layout(jnp.concatenate([src, pad_src]))
    dstp = layout(jnp.concatenate([dst, pad_dst]))
    return srcp, dstp


def _sc_counts(dsts, ones_w, zeros16):
    mesh = plsc.VectorSubcoreMesh(core_axis_name="c", subcore_axis_name="s")
    n = len(dsts)
    out_type = [jax.ShapeDtypeStruct((NC, NACC, 16), jnp.float32)
                for _ in range(n)]
    scratch_types = [
        pltpu.VMEM((NWIN0, W), jnp.int32),
        pltpu.VMEM((W, 16), jnp.float32),
        pltpu.VMEM((NSLICE, 16), jnp.float32),
        pltpu.VMEM_SHARED((NACC, 16), jnp.float32),
    ]

    @functools.partial(pl.kernel, mesh=mesh, out_type=out_type,
                       scratch_types=scratch_types,
                       compiler_params=_SC_PARAMS)
    def k(ones_ref, zeros_ref, *refs):
        d_refs = refs[0:n]
        out_refs = refs[n:2 * n]
        dbuf, ones_v, zbuf, acc_sh = refs[2 * n:]
        cid = lax.axis_index("c")
        sid = lax.axis_index("s")
        wid = cid * NS + sid
        pltpu.sync_copy(ones_ref, ones_v)
        pltpu.sync_copy(zeros_ref, zbuf)
        for t in range(n):
            pltpu.sync_copy(zbuf, acc_sh.at[pl.ds(sid * NSLICE, NSLICE)])
            pltpu.sync_copy(d_refs[t].at[wid], dbuf)
            plsc.subcore_barrier()

            @pl.loop(0, NWIN1)
            def _(w):
                pltpu.sync_copy(ones_v, acc_sh.at[dbuf.at[w]], add=True)

            @pl.when(cid == 0)
            def _():
                @pl.loop(NWIN1, NWIN0)
                def _(w):
                    pltpu.sync_copy(ones_v, acc_sh.at[dbuf.at[w]], add=True)

            plsc.subcore_barrier()
            pltpu.sync_copy(acc_sh.at[pl.ds(sid * NSLICE, NSLICE)],
                            out_refs[t].at[cid].at[pl.ds(sid * NSLICE, NSLICE)])
            if t + 1 < n:
                plsc.subcore_barrier()

    return k(ones_w, zeros16, *dsts)


def _sc_segsum5(hs, srcs, dsts, zeros32):
    mesh = plsc.VectorSubcoreMesh(core_axis_name="c", subcore_axis_name="s")
    n = len(hs)
    out_type = [jax.ShapeDtypeStruct((NC, NACC, HID), jnp.float32)
                for _ in range(n)]
    scratch_types = [
        pltpu.VMEM((NWIN0, W), jnp.int32),
        pltpu.VMEM((NWIN0, W), jnp.int32),
        pltpu.VMEM((W, HID), jnp.float32),
        pltpu.VMEM((NSLICE, HID), jnp.float32),
        pltpu.VMEM_SHARED((NACC, HID), jnp.float32),
    ]

    @functools.partial(pl.kernel, mesh=mesh, out_type=out_type,
                       scratch_types=scratch_types,
                       compiler_params=_SC_PARAMS)
    def k(zeros_ref, *refs):
        h_refs = refs[0:n]
        s_refs = refs[n:2 * n]
        d_refs = refs[2 * n:3 * n]
        out_refs = refs[3 * n:4 * n]
        sbuf, dbuf, rows, zbuf, acc_sh = refs[4 * n:]
        cid = lax.axis_index("c")
        sid = lax.axis_index("s")
        wid = cid * NS + sid
        pltpu.sync_copy(zeros_ref, zbuf)
        for c in range(n):
            pltpu.sync_copy(zbuf, acc_sh.at[pl.ds(sid * NSLICE, NSLICE)])
            pltpu.sync_copy(s_refs[c].at[wid], sbuf)
            pltpu.sync_copy(d_refs[c].at[wid], dbuf)
            plsc.subcore_barrier()

            @pl.loop(0, NWIN1)
            def _(w):
                pltpu.sync_copy(h_refs[c].at[sbuf.at[w]], rows)
                pltpu.sync_copy(rows, acc_sh.at[dbuf.at[w]], add=True)

            @pl.when(cid == 0)
            def _():
                @pl.loop(NWIN1, NWIN0)
                def _(w):
                    pltpu.sync_copy(h_refs[c].at[sbuf.at[w]], rows)
                    pltpu.sync_copy(rows, acc_sh.at[dbuf.at[w]], add=True)

            plsc.subcore_barrier()
            pltpu.sync_copy(acc_sh.at[pl.ds(sid * NSLICE, NSLICE)],
                            out_refs[c].at[cid].at[pl.ds(sid * NSLICE, NSLICE)])
            if c + 1 < n:
                plsc.subcore_barrier()

    return k(zeros32, *hs, *srcs, *dsts)


def _tc_project(xs, Wls, Wrs, brs, din):
    def body(*refs):
        x = refs[0:3]
        wl = refs[3:8]
        wr = refs[8:13]
        br = refs[13:18]
        h_out = refs[18:23]
        r_out = refs[23:28]
        for c in range(5):
            h_out[c][...] = jnp.dot(x[SRC_OF[c]][...], wl[c][...],
                                    precision=_HI)
            r_out[c][...] = jnp.dot(x[DST_OF[c]][...], wr[c][...],
                                    precision=_HI) + br[c][...]

    xspec = pl.BlockSpec((BR, din), lambda g: (g, 0))
    wspec = pl.BlockSpec((din, HID), lambda g: (0, 0))
    bspec = pl.BlockSpec((1, HID), lambda g: (0, 0))
    ospec = pl.BlockSpec((BR, HID), lambda g: (g, 0))
    out_shape = [jax.ShapeDtypeStruct((N, HID), jnp.float32)] * 10
    outs = pl.pallas_call(
        body,
        grid=(N // BR,),
        in_specs=[xspec] * 3 + [wspec] * 10 + [bspec] * 5,
        out_specs=[ospec] * 10,
        out_shape=out_shape,
    )(*xs, *Wls, *Wrs, *brs)
    return outs[0:5], outs[5:10]


def _tc_combine(aggs, cnts, rs, bls, ln_g, ln_b):
    def body(*refs):
        agg = refs[0:5]
        cnt = refs[5:10]
        r = refs[10:15]
        bl = refs[15:20]
        g_ref, b_ref = refs[20], refs[21]
        outs = refs[22:25]
        acc = [None, None, None]
        for c in range(5):
            a3 = agg[c][...]
            a = a3[0] + a3[1]
            c3 = cnt[c][...]
            cn = (c3[0] + c3[1])[:, 0:1]
            t = a / jnp.maximum(cn, 1.0) + bl[c][...] + r[c][...]
            nrm = jnp.sqrt(jnp.sum(t * t, axis=-1, keepdims=True))
            t = t / jnp.maximum(nrm, 1e-12)
            d = DST_OF[c]
            slot = {1: 0, 2: 1, 0: 2}[d]
            acc[slot] = t if acc[slot] is None else acc[slot] + t
        gv = g_ref[...]
        bv = b_ref[...]
        for i in range(3):
            v = acc[i]
            v = jnp.where(v > 0, v, jnp.exp(v) - 1.0)
            m = jnp.mean(v, axis=-1, keepdims=True)
            dlt = v - m
            var = jnp.mean(dlt * dlt, axis=-1, keepdims=True)
            outs[i][...] = dlt / jnp.sqrt(var + 1e-5) * gv + bv

    aspec = pl.BlockSpec((NC, BR, HID), lambda g: (0, g, 0))
    cspec = pl.BlockSpec((NC, BR, 16), lambda g: (0, g, 0))
    rspec = pl.BlockSpec((BR, HID), lambda g: (g, 0))
    bspec = pl.BlockSpec((1, HID), lambda g: (0, 0))
    ospec = pl.BlockSpec((BR, HID), lambda g: (g, 0))
    out_shape = [jax.ShapeDtypeStruct((N, HID), jnp.float32)] * 3
    return pl.pallas_call(
        body,
        grid=(N // BR,),
        in_specs=[aspec] * 5 + [cspec] * 5 + [rspec] * 5 + [bspec] * 7,
        out_specs=[ospec] * 3,
        out_shape=out_shape,
    )(*aggs, *cnts, *rs, *bls, ln_g, ln_b)


def _tc_final(x_fn, batch2d, Wa, ba, mu_w, pi_w):
    def body(x_ref, b_ref, wa_ref, ba_ref,
             mw1, mb1, mw2, mb2, mw3, mb3,
             pw1, pb1, pw2, pb2, pw3, pb3,
             lcb_ref, ucb_ref, mu_ref):
        x = x_ref[...]
        s = jnp.dot(x, wa_ref[...], precision=_HI) + ba_ref[...]
        m = jnp.max(s, axis=0, keepdims=True)
        e = jnp.exp(s - m)
        att = e / jnp.sum(e, axis=0, keepdims=True)
        sel = (lax.broadcasted_iota(jnp.int32, (G, N), 0)
               == b_ref[...]).astype(jnp.float32)
        bm = jnp.concatenate([x * att[:, i:i + 1] for i in range(HEADS)],
                             axis=1)
        feat = jnp.dot(sel, bm, precision=_HI)

        def mlp(z, w1, b1, w2, b2, w3, b3):
            z = jnp.dot(z, w1[...], precision=_HI) + b1[...]
            z = jnp.where(z > 0, z, jnp.exp(z) - 1.0)
            z = jnp.dot(z, w2[...], precision=_HI) + b2[...]
            z = jnp.where(z > 0, z, jnp.exp(z) - 1.0)
            return jnp.dot(z, w3[...], precision=_HI) + b3[...]

        mu = mlp(feat, mw1, mb1, mw2, mb2, mw3, mb3)
        pi = mlp(feat, pw1, pb1, pw2, pb2, pw3, pb3)
        mu_ref[...] = mu.reshape(1, G)
        lcb_ref[...] = mu.reshape(1, G) - jnp.exp(pi[:, 0].reshape(1, G))
        ucb_ref[...] = mu.reshape(1, G) + jnp.exp(pi[:, 1].reshape(1, G))

    def full(shape):
        nd = len(shape)
        return pl.BlockSpec(shape, lambda: (0,) * nd)

    (mw1, mb1), (mw2, mb2), (mw3, mb3) = mu_w
    (pw1, pb1), (pw2, pb2), (pw3, pb3) = pi_w
    args = [x_fn, batch2d, Wa, ba.reshape(1, HEADS),
            mw1, mb1.reshape(1, -1), mw2, mb2.reshape(1, -1),
            mw3, mb3.reshape(1, -1),
            pw1, pb1.reshape(1, -1), pw2, pb2.reshape(1, -1),
            pw3, pb3.reshape(1, -1)]
    in_specs = [full(a.shape) for a in args]
    out_shape = [jax.ShapeDtypeStruct((1, G), jnp.float32)] * 3
    out_specs = [full((1, G))] * 3
    lcb, ucb, mu = pl.pallas_call(
        body,
        in_specs=in_specs,
        out_specs=out_specs,
        out_shape=out_shape,
    )(*args)
    return lcb.reshape(G), ucb.reshape(G), mu.reshape(G, 1)


def kernel(x_input, x_function, x_output, edge_index_input_function,
           edge_index_function_function, edge_index_function_output,
           edge_index_output_function, edge_index_function_input,
           batch, params):
    eis = [edge_index_input_function, edge_index_function_function,
           edge_index_function_output, edge_index_output_function,
           edge_index_function_input]
    preps = [_prep_edges(e) for e in eis]
    srcs = [p[0] for p in preps]
    dsts = [p[1] for p in preps]

    ones_w = jnp.ones((W, 16), jnp.float32)
    zeros16 = jnp.zeros((NSLICE, 16), jnp.float32)
    zeros32 = jnp.zeros((NSLICE, HID), jnp.float32)
    cnts = _sc_counts(dsts, ones_w, zeros16)

    ln_g = params["ln_g"].reshape(1, HID)
    ln_b = params["ln_b"].reshape(1, HID)
    xs = [x_input.astype(jnp.float32), x_function.astype(jnp.float32),
          x_output.astype(jnp.float32)]
    for l in range(NUM_LAYERS):
        lp = params["convs"][l]
        Wls = [lp[k]["Wl"] for k in CONV_KEYS]
        bls = [lp[k]["bl"].reshape(1, HID) for k in CONV_KEYS]
        Wrs = [lp[k]["Wr"] for k in CONV_KEYS]
        brs = [lp[k]["br"].reshape(1, HID) for k in CONV_KEYS]
        din = D_FEAT if l == 0 else HID
        hs, rs = _tc_project(xs, Wls, Wrs, brs, din)
        aggs = _sc_segsum5(list(hs), srcs, dsts, zeros32)
        xfn, xout, xin = _tc_combine(list(aggs), cnts, list(rs), bls,
                                     ln_g, ln_b)
        xs = [xin, xfn, xout]

    batch2d = batch.astype(jnp.int32).reshape(1, N)
    Wa, ba = params["att"]
    return _tc_final(xs[1], batch2d, Wa, ba, params["mu"], params["pi"])
